# baseline (speedup 1.0000x reference)
import functools

import jax
import jax.numpy as jnp
from jax import lax
from jax.experimental import pallas as pl
from jax.experimental.pallas import tpu as pltpu

_INTERPRET = False


def _dot(a, b):
    return lax.dot_general(a, b, (((1,), (0,)), ((), ())),
                           preferred_element_type=jnp.float32)


def _elu(x):
    return jnp.where(x > 0, x, jnp.exp(jnp.minimum(x, 0.0)) - 1.0)


def _xw_kernel(x_ref, w_ref, out_ref):
    out_ref[0] = _dot(x_ref[...], w_ref[0])


def _xw(x, W):
    kk, fin, fout = W.shape
    n = x.shape[0]
    return pl.pallas_call(
        _xw_kernel,
        grid=(kk,),
        in_specs=[
            pl.BlockSpec((n, fin), lambda k: (0, 0)),
            pl.BlockSpec((1, fin, fout), lambda k: (k, 0, 0)),
        ],
        out_specs=pl.BlockSpec((1, n, fout), lambda k: (k, 0, 0)),
        out_shape=jax.ShapeDtypeStruct((kk, n, fout), jnp.float32),
        interpret=_INTERPRET,
    )(x, W)


def _pair_geometry(pos_d_ref, pos_sT_ref, i, j, ti, tj, r):
    pd = pos_d_ref[...]
    ps = pos_sT_ref[...]
    dx = pd[:, 0:1] - ps[0:1, :]
    dy = pd[:, 1:2] - ps[1:2, :]
    dz = pd[:, 2:3] - ps[2:3, :]
    d2 = dx * dx + dy * dy + dz * dz
    ig = i * ti + lax.broadcasted_iota(jnp.int32, (ti, tj), 0)
    jg = j * tj + lax.broadcasted_iota(jnp.int32, (ti, tj), 1)
    maskf = ((d2 < r * r) & (ig != jg)).astype(jnp.float32)
    bs = []
    for dd in (dx, dy, dz):
        u = jnp.clip(dd * (0.5 / r) + 0.5, 0.0, 1.0)
        v = u * 4.0
        i0 = jnp.clip(jnp.floor(v), 0.0, 3.0)
        frac = (v - i0)[None]
        i0i = i0.astype(jnp.int32)[None]
        c = lax.broadcasted_iota(jnp.int32, (5, ti, tj), 0)
        bs.append(jnp.where(c == i0i, 1.0 - frac, 0.0)
                  + jnp.where(c == i0i + 1, frac, 0.0))
    bs[0] = bs[0] * maskf[None]
    return maskf, bs


def _l1_kernel(pos_d_ref, pos_sT_ref, gp_ref, root_ref, b_ref, out_ref,
               s_ref, cnt_ref, *, r, ti, tj, j_tiles):
    i = pl.program_id(0)
    j = pl.program_id(1)

    @pl.when(j == 0)
    def _():
        s_ref[...] = jnp.zeros_like(s_ref)
        cnt_ref[...] = jnp.zeros_like(cnt_ref)

    maskf, (b0, b1, b2) = _pair_geometry(pos_d_ref, pos_sT_ref, i, j, ti, tj, r)
    cnt_ref[...] += jnp.sum(maskf, axis=1, keepdims=True)

    chunks = []
    for k0 in range(5):
        for k1 in range(5):
            p01 = (b0[k0] * b1[k1])[None]
            chunks.append(jnp.sum(b2 * p01, axis=2))
    chunks.append(jnp.zeros((3, ti), jnp.float32))
    s_ref[...] += jnp.concatenate(chunks, axis=0)

    @pl.when(j == j_tiles - 1)
    def _():
        agg = lax.dot_general(s_ref[...], gp_ref[...],
                              (((0,), (0,)), ((), ())),
                              preferred_element_type=jnp.float32)
        agg = agg / jnp.maximum(cnt_ref[...], 1.0)
        out_ref[...] = _elu(agg + root_ref[...] + b_ref[...])


def _l1_layer(pos_pad, posT, W1, root1, b1, r, ti=256, tj=128):
    n = pos_pad.shape[0]
    fout = W1.shape[2]
    i_tiles, j_tiles = n // ti, n // tj
    gp = jnp.concatenate([W1[:, 0, :], jnp.zeros((3, fout), jnp.float32)], 0)
    body = functools.partial(_l1_kernel, r=r, ti=ti, tj=tj, j_tiles=j_tiles)
    return pl.pallas_call(
        body,
        grid=(i_tiles, j_tiles),
        in_specs=[
            pl.BlockSpec((ti, 8), lambda i, j: (i, 0)),
            pl.BlockSpec((8, tj), lambda i, j: (0, j)),
            pl.BlockSpec((128, fout), lambda i, j: (0, 0)),
            pl.BlockSpec((1, fout), lambda i, j: (0, 0)),
            pl.BlockSpec((1, fout), lambda i, j: (0, 0)),
        ],
        out_specs=pl.BlockSpec((ti, fout), lambda i, j: (i, 0)),
        out_shape=jax.ShapeDtypeStruct((n, fout), jnp.float32),
        scratch_shapes=[pltpu.VMEM((128, ti), jnp.float32),
                        pltpu.VMEM((ti, 1), jnp.float32)],
        compiler_params=pltpu.CompilerParams(
            dimension_semantics=("arbitrary", "arbitrary")),
        interpret=_INTERPRET,
    )(pos_pad, posT, gp, root1.reshape(1, fout), b1.reshape(1, fout))


def _layer_kernel(pos_d_ref, pos_sT_ref, xw_ref, xd_ref, root_ref, b_ref,
                  out_ref, acc_ref, cnt_ref, *, r, ti, tj, j_tiles, o_dim):
    i = pl.program_id(0)
    j = pl.program_id(1)

    @pl.when(j == 0)
    def _():
        acc_ref[...] = jnp.zeros_like(acc_ref)
        cnt_ref[...] = jnp.zeros_like(cnt_ref)

    maskf, (b0, b1, b2) = _pair_geometry(pos_d_ref, pos_sT_ref, i, j, ti, tj, r)
    cnt_ref[...] += jnp.sum(maskf, axis=1, keepdims=True)

    acc = jnp.zeros((ti, o_dim), jnp.float32)
    for k0 in range(5):
        for k1 in range(5):
            p01 = b0[k0] * b1[k1]
            for k2 in range(5):
                acc += _dot(p01 * b2[k2], xw_ref[k0 * 25 + k1 * 5 + k2])
    acc_ref[...] += acc

    @pl.when(j == j_tiles - 1)
    def _():
        agg = acc_ref[...] / jnp.maximum(cnt_ref[...], 1.0)
        res = agg + _dot(xd_ref[...], root_ref[...]) + b_ref[...]
        out_ref[...] = _elu(res)


def _spline_layer(pos_pad, posT, x, W, root, b, r, xw_const=None,
                  ti=256, tj=128):
    n = pos_pad.shape[0]
    fin, fout = root.shape
    i_tiles, j_tiles = n // ti, n // tj
    if xw_const is not None:
        xw = jnp.broadcast_to(xw_const[:, None, :], (125, tj, fout))
        xw_spec = pl.BlockSpec((125, tj, fout), lambda i, j: (0, 0, 0))
    else:
        xw = _xw(x, W)
        xw_spec = pl.BlockSpec((125, tj, fout), lambda i, j: (0, j, 0))
    body = functools.partial(_layer_kernel, r=r, ti=ti, tj=tj,
                             j_tiles=j_tiles, o_dim=fout)
    return pl.pallas_call(
        body,
        grid=(i_tiles, j_tiles),
        in_specs=[
            pl.BlockSpec((ti, 8), lambda i, j: (i, 0)),
            pl.BlockSpec((8, tj), lambda i, j: (0, j)),
            xw_spec,
            pl.BlockSpec((ti, fin), lambda i, j: (i, 0)),
            pl.BlockSpec((fin, fout), lambda i, j: (0, 0)),
            pl.BlockSpec((1, fout), lambda i, j: (0, 0)),
        ],
        out_specs=pl.BlockSpec((ti, fout), lambda i, j: (i, 0)),
        out_shape=jax.ShapeDtypeStruct((n, fout), jnp.float32),
        scratch_shapes=[pltpu.VMEM((ti, fout), jnp.float32),
                        pltpu.VMEM((ti, 1), jnp.float32)],
        compiler_params=pltpu.CompilerParams(
            dimension_semantics=("arbitrary", "arbitrary")),
        interpret=_INTERPRET,
    )(pos_pad, posT, xw, x, root, b.reshape(1, fout))


def _fps_kernel(posT_ref, pos_ref, x_ref, pos_sel_ref, x_sel_ref, dist_ref,
                *, n, m):
    col = lax.broadcasted_iota(jnp.int32, (1, n), 1)
    x0 = posT_ref[0:1, :]
    y0 = posT_ref[1:2, :]
    z0 = posT_ref[2:3, :]
    dist_ref[...] = jnp.full((1, n), jnp.inf, jnp.float32)

    def body(it, cur):
        pos_sel_ref[pl.ds(it, 1), :] = pos_ref[pl.ds(cur, 1), :]
        x_sel_ref[pl.ds(it, 1), :] = x_ref[pl.ds(cur, 1), :]
        cm = col == cur
        px = jnp.sum(jnp.where(cm, x0, 0.0), axis=1, keepdims=True)
        py = jnp.sum(jnp.where(cm, y0, 0.0), axis=1, keepdims=True)
        pz = jnp.sum(jnp.where(cm, z0, 0.0), axis=1, keepdims=True)
        d = (x0 - px) ** 2 + (y0 - py) ** 2 + (z0 - pz) ** 2
        dist = jnp.minimum(dist_ref[...], d)
        dist_ref[...] = dist
        mx = jnp.max(dist)
        return jnp.min(jnp.where(dist == mx, col, n)).astype(jnp.int32)

    lax.fori_loop(0, m, body, jnp.int32(0))


def _fps(pos_pad, posT, x, m):
    n, f = x.shape
    body = functools.partial(_fps_kernel, n=n, m=m)
    return pl.pallas_call(
        body,
        in_specs=[
            pl.BlockSpec((8, n), lambda: (0, 0)),
            pl.BlockSpec((n, 8), lambda: (0, 0)),
            pl.BlockSpec((n, f), lambda: (0, 0)),
        ],
        out_specs=[
            pl.BlockSpec((m, 8), lambda: (0, 0)),
            pl.BlockSpec((m, f), lambda: (0, 0)),
        ],
        out_shape=[jax.ShapeDtypeStruct((m, 8), jnp.float32),
                   jax.ShapeDtypeStruct((m, f), jnp.float32)],
        scratch_shapes=[pltpu.VMEM((1, n), jnp.float32)],
        interpret=_INTERPRET,
    )(posT, pos_pad, x)


def _head_kernel(x_ref, w1_ref, b1_ref, w2_ref, b2_ref, w3_ref, b3_ref,
                 out_ref, *, n):
    h = jnp.sum(x_ref[...], axis=0, keepdims=True) * (1.0 / n)
    h = _elu(_dot(h, w1_ref[...]) + b1_ref[...])
    h = _elu(_dot(h, w2_ref[...]) + b2_ref[...])
    o = _dot(h, w3_ref[...]) + b3_ref[...]
    mx = jnp.max(o)
    out_ref[...] = o - mx - jnp.log(jnp.sum(jnp.exp(o - mx)))


def _head(x, lw1, lb1, lw2, lb2, lw3, lb3):
    n, f = x.shape
    body = functools.partial(_head_kernel, n=n)
    return pl.pallas_call(
        body,
        out_shape=jax.ShapeDtypeStruct((1, 10), jnp.float32),
        interpret=_INTERPRET,
    )(x, lw1, lb1.reshape(1, -1), lw2, lb2.reshape(1, -1),
      lw3, lb3.reshape(1, -1))


def _pad8(p):
    n = p.shape[0]
    return jnp.concatenate([p, jnp.zeros((n, 5), jnp.float32)], axis=1)


def kernel(pos, batch, W1, root1, b1, W2, root2, b2, W3, root3, b3,
           lw1, lb1, lw2, lb2, lw3, lb3):
    n = pos.shape[0]
    pos_pad = _pad8(pos)
    posT = pos_pad.T

    x1 = _l1_layer(pos_pad, posT, W1, root1, b1, r=0.2)

    m1 = n // 2
    pos1_pad, x1s = _fps(pos_pad, posT, x1, m1)
    pos1T = pos1_pad.T

    x2 = _spline_layer(pos1_pad, pos1T, x1s, W2, root2, b2, r=0.4)

    m2 = m1 // 4
    pos2_pad, x2s = _fps(pos1_pad, pos1T, x2, m2)
    pos2T = pos2_pad.T

    x3 = _spline_layer(pos2_pad, pos2T, x2s, W3, root3, b3, r=1.0)

    return _head(x3, lw1, lb1, lw2, lb2, lw3, lb3)


# speedup vs baseline: 1.3729x; 1.3729x over previous
import functools

import jax
import jax.numpy as jnp
from jax import lax
from jax.experimental import pallas as pl
from jax.experimental.pallas import tpu as pltpu

_INTERPRET = False


def _dot(a, b):
    return lax.dot_general(a, b, (((1,), (0,)), ((), ())),
                           preferred_element_type=jnp.float32)


def _elu(x):
    return jnp.where(x > 0, x, jnp.exp(jnp.minimum(x, 0.0)) - 1.0)


def _xw_kernel(x_ref, w_ref, out_ref):
    out_ref[0] = _dot(x_ref[...], w_ref[0]).astype(jnp.bfloat16)


def _xw(x, W):
    kk, fin, fout = W.shape
    n = x.shape[0]
    return pl.pallas_call(
        _xw_kernel,
        grid=(kk,),
        in_specs=[
            pl.BlockSpec((n, fin), lambda k: (0, 0)),
            pl.BlockSpec((1, fin, fout), lambda k: (k, 0, 0)),
        ],
        out_specs=pl.BlockSpec((1, n, fout), lambda k: (k, 0, 0)),
        out_shape=jax.ShapeDtypeStruct((kk, n, fout), jnp.bfloat16),
        interpret=_INTERPRET,
    )(x, W)


def _pair_geometry(pos_d_ref, pos_sT_ref, i, j, ti, tj, r):
    pd = pos_d_ref[...]
    ps = pos_sT_ref[...]
    dx = pd[:, 0:1] - ps[0:1, :]
    dy = pd[:, 1:2] - ps[1:2, :]
    dz = pd[:, 2:3] - ps[2:3, :]
    d2 = dx * dx + dy * dy + dz * dz
    ig = i * ti + lax.broadcasted_iota(jnp.int32, (ti, tj), 0)
    jg = j * tj + lax.broadcasted_iota(jnp.int32, (ti, tj), 1)
    maskf = ((d2 < r * r) & (ig != jg)).astype(jnp.float32)
    bs = []
    for dd in (dx, dy, dz):
        u = jnp.clip(dd * (0.5 / r) + 0.5, 0.0, 1.0)
        v = u * 4.0
        i0 = jnp.clip(jnp.floor(v), 0.0, 3.0)
        frac = (v - i0)[None]
        i0i = i0.astype(jnp.int32)[None]
        c = lax.broadcasted_iota(jnp.int32, (5, ti, tj), 0)
        bs.append(jnp.where(c == i0i, 1.0 - frac, 0.0)
                  + jnp.where(c == i0i + 1, frac, 0.0))
    bs[0] = bs[0] * maskf[None]
    return maskf, bs


def _layer_kernel(pos_d_ref, pos_sT_ref, xw_ref, xd_ref, root_ref, b_ref,
                  out_ref, acc_ref, cnt_ref, *, r, ti, tj, j_tiles, o_dim):
    i = pl.program_id(0)
    j = pl.program_id(1)

    @pl.when(j == 0)
    def _():
        acc_ref[...] = jnp.zeros_like(acc_ref)
        cnt_ref[...] = jnp.zeros_like(cnt_ref)

    maskf, (b0, b1, b2) = _pair_geometry(pos_d_ref, pos_sT_ref, i, j, ti, tj, r)
    cnt_ref[...] += jnp.sum(maskf, axis=1, keepdims=True)

    b0 = b0.astype(jnp.bfloat16)
    b1 = b1.astype(jnp.bfloat16)
    b2 = b2.astype(jnp.bfloat16)
    acc = jnp.zeros((ti, o_dim), jnp.float32)
    for k0 in range(5):
        for k1 in range(5):
            p01 = b0[k0] * b1[k1]
            for k2 in range(5):
                acc += _dot(p01 * b2[k2], xw_ref[k0 * 25 + k1 * 5 + k2])
    acc_ref[...] += acc

    @pl.when(j == j_tiles - 1)
    def _():
        agg = acc_ref[...] / jnp.maximum(cnt_ref[...], 1.0)
        res = agg + _dot(xd_ref[...], root_ref[...]) + b_ref[...]
        out_ref[...] = _elu(res)


def _spline_layer(pos_pad, posT, x, W, root, b, r, xw_const=None,
                  ti=256, tj=128):
    n = pos_pad.shape[0]
    fin, fout = root.shape
    i_tiles, j_tiles = n // ti, n // tj
    if xw_const is not None:
        xw = jnp.broadcast_to(xw_const.astype(jnp.bfloat16)[:, None, :],
                              (125, tj, fout))
        xw_spec = pl.BlockSpec((125, tj, fout), lambda i, j: (0, 0, 0))
    else:
        xw = _xw(x, W)
        xw_spec = pl.BlockSpec((125, tj, fout), lambda i, j: (0, j, 0))
    body = functools.partial(_layer_kernel, r=r, ti=ti, tj=tj,
                             j_tiles=j_tiles, o_dim=fout)
    return pl.pallas_call(
        body,
        grid=(i_tiles, j_tiles),
        in_specs=[
            pl.BlockSpec((ti, 8), lambda i, j: (i, 0)),
            pl.BlockSpec((8, tj), lambda i, j: (0, j)),
            xw_spec,
            pl.BlockSpec((ti, fin), lambda i, j: (i, 0)),
            pl.BlockSpec((fin, fout), lambda i, j: (0, 0)),
            pl.BlockSpec((1, fout), lambda i, j: (0, 0)),
        ],
        out_specs=pl.BlockSpec((ti, fout), lambda i, j: (i, 0)),
        out_shape=jax.ShapeDtypeStruct((n, fout), jnp.float32),
        scratch_shapes=[pltpu.VMEM((ti, fout), jnp.float32),
                        pltpu.VMEM((ti, 1), jnp.float32)],
        compiler_params=pltpu.CompilerParams(
            dimension_semantics=("arbitrary", "arbitrary")),
        interpret=_INTERPRET,
    )(pos_pad, posT, xw, x, root, b.reshape(1, fout))


def _fps_kernel(pos8_ref, pos_ref, x_ref, pos_sel_ref, x_sel_ref, dist_ref,
                *, n, m, nc):
    colidx = (lax.broadcasted_iota(jnp.int32, (8, nc), 0) * nc
              + lax.broadcasted_iota(jnp.int32, (8, nc), 1))
    xs = pos8_ref[0:8, :]
    ys = pos8_ref[8:16, :]
    zs = pos8_ref[16:24, :]
    dist_ref[...] = jnp.full((8, nc), jnp.inf, jnp.float32)

    def body(it, cur):
        pos_sel_ref[pl.ds(it, 1), :] = pos_ref[pl.ds(cur, 1), :]
        x_sel_ref[pl.ds(it, 1), :] = x_ref[pl.ds(cur, 1), :]
        cm = colidx == cur
        px = jnp.sum(jnp.where(cm, xs, 0.0))
        py = jnp.sum(jnp.where(cm, ys, 0.0))
        pz = jnp.sum(jnp.where(cm, zs, 0.0))
        d = (xs - px) ** 2 + (ys - py) ** 2 + (zs - pz) ** 2
        dist = jnp.minimum(dist_ref[...], d)
        dist_ref[...] = dist
        mx = jnp.max(dist)
        return jnp.min(jnp.where(dist == mx, colidx, n)).astype(jnp.int32)

    lax.fori_loop(0, m, body, jnp.int32(0))


def _fps(pos_pad, posT, x, m):
    n, f = x.shape
    nc = n // 8
    pos8 = jnp.concatenate([posT[0].reshape(8, nc), posT[1].reshape(8, nc),
                            posT[2].reshape(8, nc)], axis=0)
    body = functools.partial(_fps_kernel, n=n, m=m, nc=nc)
    return pl.pallas_call(
        body,
        in_specs=[
            pl.BlockSpec((24, nc), lambda: (0, 0)),
            pl.BlockSpec((n, 8), lambda: (0, 0)),
            pl.BlockSpec((n, f), lambda: (0, 0)),
        ],
        out_specs=[
            pl.BlockSpec((m, 8), lambda: (0, 0)),
            pl.BlockSpec((m, f), lambda: (0, 0)),
        ],
        out_shape=[jax.ShapeDtypeStruct((m, 8), jnp.float32),
                   jax.ShapeDtypeStruct((m, f), jnp.float32)],
        scratch_shapes=[pltpu.VMEM((8, nc), jnp.float32)],
        interpret=_INTERPRET,
    )(pos8, pos_pad, x)


def _head_kernel(x_ref, w1_ref, b1_ref, w2_ref, b2_ref, w3_ref, b3_ref,
                 out_ref, *, n):
    h = jnp.sum(x_ref[...], axis=0, keepdims=True) * (1.0 / n)
    h = _elu(_dot(h, w1_ref[...]) + b1_ref[...])
    h = _elu(_dot(h, w2_ref[...]) + b2_ref[...])
    o = _dot(h, w3_ref[...]) + b3_ref[...]
    mx = jnp.max(o)
    out_ref[...] = o - mx - jnp.log(jnp.sum(jnp.exp(o - mx)))


def _head(x, lw1, lb1, lw2, lb2, lw3, lb3):
    n, f = x.shape
    body = functools.partial(_head_kernel, n=n)
    return pl.pallas_call(
        body,
        out_shape=jax.ShapeDtypeStruct((1, 10), jnp.float32),
        interpret=_INTERPRET,
    )(x, lw1, lb1.reshape(1, -1), lw2, lb2.reshape(1, -1),
      lw3, lb3.reshape(1, -1))


def _pad8(p):
    n = p.shape[0]
    return jnp.concatenate([p, jnp.zeros((n, 5), jnp.float32)], axis=1)


def kernel(pos, batch, W1, root1, b1, W2, root2, b2, W3, root3, b3,
           lw1, lb1, lw2, lb2, lw3, lb3):
    n = pos.shape[0]
    pos_pad = _pad8(pos)
    posT = pos_pad.T

    ones = jnp.ones((n, 1), jnp.float32)
    x1 = _spline_layer(pos_pad, posT, ones, W1, root1, b1, r=0.2,
                       xw_const=W1[:, 0, :])

    m1 = n // 2
    pos1_pad, x1s = _fps(pos_pad, posT, x1, m1)
    pos1T = pos1_pad.T

    x2 = _spline_layer(pos1_pad, pos1T, x1s, W2, root2, b2, r=0.4)

    m2 = m1 // 4
    pos2_pad, x2s = _fps(pos1_pad, pos1T, x2, m2)
    pos2T = pos2_pad.T

    x3 = _spline_layer(pos2_pad, pos2T, x2s, W3, root3, b3, r=1.0)

    return _head(x3, lw1, lb1, lw2, lb2, lw3, lb3)


# speedup vs baseline: 1.7138x; 1.2483x over previous
import functools

import jax
import jax.numpy as jnp
from jax import lax
from jax.experimental import pallas as pl
from jax.experimental.pallas import tpu as pltpu

_INTERPRET = False


def _dot(a, b):
    return lax.dot_general(a, b, (((1,), (0,)), ((), ())),
                           preferred_element_type=jnp.float32)


def _elu(x):
    return jnp.where(x > 0, x, jnp.exp(jnp.minimum(x, 0.0)) - 1.0)


def _xw_kernel(x_ref, w_ref, out_ref):
    out_ref[0] = _dot(x_ref[...], w_ref[0]).astype(jnp.bfloat16)


def _xw(x, W):
    kk, fin, fout = W.shape
    n = x.shape[0]
    return pl.pallas_call(
        _xw_kernel,
        grid=(kk,),
        in_specs=[
            pl.BlockSpec((n, fin), lambda k: (0, 0)),
            pl.BlockSpec((1, fin, fout), lambda k: (k, 0, 0)),
        ],
        out_specs=pl.BlockSpec((1, n, fout), lambda k: (k, 0, 0)),
        out_shape=jax.ShapeDtypeStruct((kk, n, fout), jnp.bfloat16),
        interpret=_INTERPRET,
    )(x, W)


def _pair_geometry(pos_d_ref, pos_sT_ref, i, j, ti, tj, r):
    pd = pos_d_ref[...]
    ps = pos_sT_ref[...]
    dx = pd[:, 0:1] - ps[0:1, :]
    dy = pd[:, 1:2] - ps[1:2, :]
    dz = pd[:, 2:3] - ps[2:3, :]
    d2 = dx * dx + dy * dy + dz * dz
    ig = i * ti + lax.broadcasted_iota(jnp.int32, (ti, tj), 0)
    jg = j * tj + lax.broadcasted_iota(jnp.int32, (ti, tj), 1)
    maskf = ((d2 < r * r) & (ig != jg)).astype(jnp.float32)
    bs = []
    for dd in (dx, dy, dz):
        u = jnp.clip(dd * (0.5 / r) + 0.5, 0.0, 1.0)
        v = u * 4.0
        i0 = jnp.clip(jnp.floor(v), 0.0, 3.0)
        frac = (v - i0)[None]
        i0i = i0.astype(jnp.int32)[None]
        c = lax.broadcasted_iota(jnp.int32, (5, ti, tj), 0)
        bs.append(jnp.where(c == i0i, 1.0 - frac, 0.0)
                  + jnp.where(c == i0i + 1, frac, 0.0))
    bs[0] = bs[0] * maskf[None]
    return maskf, bs


def _layer_kernel(pos_d_ref, pos_sT_ref, xw_ref, xd_ref, root_ref, b_ref,
                  out_ref, acc_ref, cnt_ref, *, r, ti, tj, j_tiles, o_dim):
    i = pl.program_id(0)
    j = pl.program_id(1)

    @pl.when(j == 0)
    def _():
        acc_ref[...] = jnp.zeros_like(acc_ref)
        cnt_ref[...] = jnp.zeros_like(cnt_ref)

    zd = pos_d_ref[...][:, 2:3]
    zs = pos_sT_ref[...][2:3, :]
    gap = jnp.maximum(jnp.min(zd) - jnp.max(zs), jnp.min(zs) - jnp.max(zd))

    @pl.when(gap < r)
    def _():
        maskf, (b0, b1, b2) = _pair_geometry(pos_d_ref, pos_sT_ref,
                                             i, j, ti, tj, r)
        cnt_ref[...] += jnp.sum(maskf, axis=1, keepdims=True)

        b0 = b0.astype(jnp.bfloat16)
        b1 = b1.astype(jnp.bfloat16)
        b2 = b2.astype(jnp.bfloat16)
        acc = jnp.zeros((ti, o_dim), jnp.float32)
        for k0 in range(5):
            for k1 in range(5):
                p01 = b0[k0] * b1[k1]
                a5 = jnp.concatenate([p01 * b2[k2] for k2 in range(5)], axis=1)
                x5 = xw_ref[k0 * 25 + k1 * 5:k0 * 25 + k1 * 5 + 5]
                acc += _dot(a5, x5.reshape(5 * tj, o_dim))
        acc_ref[...] += acc

    @pl.when(j == j_tiles - 1)
    def _():
        agg = acc_ref[...] / jnp.maximum(cnt_ref[...], 1.0)
        res = agg + _dot(xd_ref[...], root_ref[...]) + b_ref[...]
        out_ref[...] = _elu(res)


def _spline_layer(pos_pad, posT, x, W, root, b, r, xw_const=None,
                  ti=256, tj=128):
    n = pos_pad.shape[0]
    fin, fout = root.shape
    i_tiles, j_tiles = n // ti, n // tj
    if xw_const is not None:
        xw = jnp.broadcast_to(xw_const.astype(jnp.bfloat16)[:, None, :],
                              (125, tj, fout))
        xw_spec = pl.BlockSpec((125, tj, fout), lambda i, j: (0, 0, 0))
    else:
        xw = _xw(x, W)
        xw_spec = pl.BlockSpec((125, tj, fout), lambda i, j: (0, j, 0))
    body = functools.partial(_layer_kernel, r=r, ti=ti, tj=tj,
                             j_tiles=j_tiles, o_dim=fout)
    return pl.pallas_call(
        body,
        grid=(i_tiles, j_tiles),
        in_specs=[
            pl.BlockSpec((ti, 8), lambda i, j: (i, 0)),
            pl.BlockSpec((8, tj), lambda i, j: (0, j)),
            xw_spec,
            pl.BlockSpec((ti, fin), lambda i, j: (i, 0)),
            pl.BlockSpec((fin, fout), lambda i, j: (0, 0)),
            pl.BlockSpec((1, fout), lambda i, j: (0, 0)),
        ],
        out_specs=pl.BlockSpec((ti, fout), lambda i, j: (i, 0)),
        out_shape=jax.ShapeDtypeStruct((n, fout), jnp.float32),
        scratch_shapes=[pltpu.VMEM((ti, fout), jnp.float32),
                        pltpu.VMEM((ti, 1), jnp.float32)],
        compiler_params=pltpu.CompilerParams(
            dimension_semantics=("arbitrary", "arbitrary")),
        interpret=_INTERPRET,
    )(pos_pad, posT, xw, x, root, b.reshape(1, fout))


def _fps_kernel(pos8_ref, pos_ref, x_ref, pos_sel_ref, x_sel_ref, dist_ref,
                *, n, m, nc):
    colidx = (lax.broadcasted_iota(jnp.int32, (8, nc), 0) * nc
              + lax.broadcasted_iota(jnp.int32, (8, nc), 1))
    xs = pos8_ref[0:8, :]
    ys = pos8_ref[8:16, :]
    zs = pos8_ref[16:24, :]
    dist_ref[...] = jnp.full((8, nc), jnp.inf, jnp.float32)

    def body(it, cur):
        pos_sel_ref[pl.ds(it, 1), :] = pos_ref[pl.ds(cur, 1), :]
        x_sel_ref[pl.ds(it, 1), :] = x_ref[pl.ds(cur, 1), :]
        cm = colidx == cur
        px = jnp.sum(jnp.where(cm, xs, 0.0))
        py = jnp.sum(jnp.where(cm, ys, 0.0))
        pz = jnp.sum(jnp.where(cm, zs, 0.0))
        d = (xs - px) ** 2 + (ys - py) ** 2 + (zs - pz) ** 2
        dist = jnp.minimum(dist_ref[...], d)
        dist_ref[...] = dist
        mx = jnp.max(dist)
        return jnp.min(jnp.where(dist == mx, colidx, n)).astype(jnp.int32)

    lax.fori_loop(0, m, body, jnp.int32(0))


def _fps(pos_pad, posT, x, m):
    n, f = x.shape
    nc = n // 8
    pos8 = jnp.concatenate([posT[0].reshape(8, nc), posT[1].reshape(8, nc),
                            posT[2].reshape(8, nc)], axis=0)
    body = functools.partial(_fps_kernel, n=n, m=m, nc=nc)
    return pl.pallas_call(
        body,
        in_specs=[
            pl.BlockSpec((24, nc), lambda: (0, 0)),
            pl.BlockSpec((n, 8), lambda: (0, 0)),
            pl.BlockSpec((n, f), lambda: (0, 0)),
        ],
        out_specs=[
            pl.BlockSpec((m, 8), lambda: (0, 0)),
            pl.BlockSpec((m, f), lambda: (0, 0)),
        ],
        out_shape=[jax.ShapeDtypeStruct((m, 8), jnp.float32),
                   jax.ShapeDtypeStruct((m, f), jnp.float32)],
        scratch_shapes=[pltpu.VMEM((8, nc), jnp.float32)],
        interpret=_INTERPRET,
    )(pos8, pos_pad, x)


def _head_kernel(x_ref, w1_ref, b1_ref, w2_ref, b2_ref, w3_ref, b3_ref,
                 out_ref, *, n):
    h = jnp.sum(x_ref[...], axis=0, keepdims=True) * (1.0 / n)
    h = _elu(_dot(h, w1_ref[...]) + b1_ref[...])
    h = _elu(_dot(h, w2_ref[...]) + b2_ref[...])
    o = _dot(h, w3_ref[...]) + b3_ref[...]
    mx = jnp.max(o)
    out_ref[...] = o - mx - jnp.log(jnp.sum(jnp.exp(o - mx)))


def _head(x, lw1, lb1, lw2, lb2, lw3, lb3):
    n, f = x.shape
    body = functools.partial(_head_kernel, n=n)
    return pl.pallas_call(
        body,
        out_shape=jax.ShapeDtypeStruct((1, 10), jnp.float32),
        interpret=_INTERPRET,
    )(x, lw1, lb1.reshape(1, -1), lw2, lb2.reshape(1, -1),
      lw3, lb3.reshape(1, -1))


def _pad8(p):
    n = p.shape[0]
    return jnp.concatenate([p, jnp.zeros((n, 5), jnp.float32)], axis=1)


def kernel(pos, batch, W1, root1, b1, W2, root2, b2, W3, root3, b3,
           lw1, lb1, lw2, lb2, lw3, lb3):
    n = pos.shape[0]
    pos_pad = _pad8(pos)
    posT = pos_pad.T

    perm0 = jnp.argsort(pos[:, 2])
    rank0 = jnp.argsort(perm0)
    pos_pad_s = pos_pad[perm0]
    ones = jnp.ones((n, 1), jnp.float32)
    x1 = _spline_layer(pos_pad_s, pos_pad_s.T, ones, W1, root1, b1, r=0.2,
                       xw_const=W1[:, 0, :])[rank0]

    m1 = n // 2
    pos1_pad, x1s = _fps(pos_pad, posT, x1, m1)

    perm1 = jnp.argsort(pos1_pad[:, 2])
    rank1 = jnp.argsort(perm1)
    pos1_pad_s = pos1_pad[perm1]
    x2 = _spline_layer(pos1_pad_s, pos1_pad_s.T, x1s[perm1], W2, root2, b2,
                       r=0.4)[rank1]

    m2 = m1 // 4
    pos2_pad, x2s = _fps(pos1_pad, pos1_pad.T, x2, m2)

    x3 = _spline_layer(pos2_pad, pos2_pad.T, x2s, W3, root3, b3, r=1.0)

    return _head(x3, lw1, lb1, lw2, lb2, lw3, lb3)


# speedup vs baseline: 1.7783x; 1.0377x over previous
import functools

import jax
import jax.numpy as jnp
from jax import lax
from jax.experimental import pallas as pl
from jax.experimental.pallas import tpu as pltpu

_INTERPRET = False


def _dot(a, b):
    return lax.dot_general(a, b, (((1,), (0,)), ((), ())),
                           preferred_element_type=jnp.float32)


def _elu(x):
    return jnp.where(x > 0, x, jnp.exp(jnp.minimum(x, 0.0)) - 1.0)


def _xw_kernel(x_ref, w_ref, out_ref):
    out_ref[0] = _dot(x_ref[...], w_ref[0]).astype(jnp.bfloat16)


def _xw(x, W):
    kk, fin, fout = W.shape
    n = x.shape[0]
    return pl.pallas_call(
        _xw_kernel,
        grid=(kk,),
        in_specs=[
            pl.BlockSpec((n, fin), lambda k: (0, 0)),
            pl.BlockSpec((1, fin, fout), lambda k: (k, 0, 0)),
        ],
        out_specs=pl.BlockSpec((1, n, fout), lambda k: (k, 0, 0)),
        out_shape=jax.ShapeDtypeStruct((kk, n, fout), jnp.bfloat16),
        interpret=_INTERPRET,
    )(x, W)


def _pair_geometry(pos_d_ref, pos_sT_ref, i, j, ti, tj, r):
    pd = pos_d_ref[...]
    ps = pos_sT_ref[...]
    dx = pd[:, 0:1] - ps[0:1, :]
    dy = pd[:, 1:2] - ps[1:2, :]
    dz = pd[:, 2:3] - ps[2:3, :]
    d2 = dx * dx + dy * dy + dz * dz
    ig = i * ti + lax.broadcasted_iota(jnp.int32, (ti, tj), 0)
    jg = j * tj + lax.broadcasted_iota(jnp.int32, (ti, tj), 1)
    maskf = ((d2 < r * r) & (ig != jg)).astype(jnp.float32)
    c = lax.broadcasted_iota(jnp.int32, (5, ti, tj), 0).astype(jnp.float32)
    bs = []
    for dd in (dx, dy, dz):
        v = jnp.clip(dd * (0.5 / r) + 0.5, 0.0, 1.0) * 4.0
        bs.append(jnp.maximum(1.0 - jnp.abs(v[None] - c), 0.0))
    bs[0] = bs[0] * maskf[None]
    return maskf, bs


def _layer_kernel(pos_d_ref, pos_sT_ref, xw_ref, xd_ref, root_ref, b_ref,
                  out_ref, acc_ref, cnt_ref, *, r, ti, tj, j_tiles, o_dim):
    i = pl.program_id(0)
    j = pl.program_id(1)

    @pl.when(j == 0)
    def _():
        acc_ref[...] = jnp.zeros_like(acc_ref)
        cnt_ref[...] = jnp.zeros_like(cnt_ref)

    zd = pos_d_ref[...][:, 2:3]
    zs = pos_sT_ref[...][2:3, :]
    gap = jnp.maximum(jnp.min(zd) - jnp.max(zs), jnp.min(zs) - jnp.max(zd))

    @pl.when(gap < r)
    def _():
        maskf, (b0, b1, b2) = _pair_geometry(pos_d_ref, pos_sT_ref,
                                             i, j, ti, tj, r)
        cnt_ref[...] += jnp.sum(maskf, axis=1, keepdims=True)

        b0 = b0.astype(jnp.bfloat16)
        b1 = b1.astype(jnp.bfloat16)
        b2 = b2.astype(jnp.bfloat16)
        acc = jnp.zeros((ti, o_dim), jnp.float32)
        for k0 in range(5):
            for k1 in range(5):
                p01 = b0[k0] * b1[k1]
                a5 = jnp.concatenate([p01 * b2[k2] for k2 in range(5)], axis=1)
                x5 = xw_ref[k0 * 25 + k1 * 5:k0 * 25 + k1 * 5 + 5]
                acc += _dot(a5, x5.reshape(5 * tj, o_dim))
        acc_ref[...] += acc

    @pl.when(j == j_tiles - 1)
    def _():
        agg = acc_ref[...] / jnp.maximum(cnt_ref[...], 1.0)
        res = agg + _dot(xd_ref[...], root_ref[...]) + b_ref[...]
        out_ref[...] = _elu(res)


def _spline_layer(pos_pad, posT, x, W, root, b, r, xw_const=None,
                  ti=256, tj=128):
    n = pos_pad.shape[0]
    fin, fout = root.shape
    i_tiles, j_tiles = n // ti, n // tj
    if xw_const is not None:
        xw = jnp.broadcast_to(xw_const.astype(jnp.bfloat16)[:, None, :],
                              (125, tj, fout))
        xw_spec = pl.BlockSpec((125, tj, fout), lambda i, j: (0, 0, 0))
    else:
        xw = _xw(x, W)
        xw_spec = pl.BlockSpec((125, tj, fout), lambda i, j: (0, j, 0))
    body = functools.partial(_layer_kernel, r=r, ti=ti, tj=tj,
                             j_tiles=j_tiles, o_dim=fout)
    return pl.pallas_call(
        body,
        grid=(i_tiles, j_tiles),
        in_specs=[
            pl.BlockSpec((ti, 8), lambda i, j: (i, 0)),
            pl.BlockSpec((8, tj), lambda i, j: (0, j)),
            xw_spec,
            pl.BlockSpec((ti, fin), lambda i, j: (i, 0)),
            pl.BlockSpec((fin, fout), lambda i, j: (0, 0)),
            pl.BlockSpec((1, fout), lambda i, j: (0, 0)),
        ],
        out_specs=pl.BlockSpec((ti, fout), lambda i, j: (i, 0)),
        out_shape=jax.ShapeDtypeStruct((n, fout), jnp.float32),
        scratch_shapes=[pltpu.VMEM((ti, fout), jnp.float32),
                        pltpu.VMEM((ti, 1), jnp.float32)],
        compiler_params=pltpu.CompilerParams(
            dimension_semantics=("arbitrary", "arbitrary")),
        interpret=_INTERPRET,
    )(pos_pad, posT, xw, x, root, b.reshape(1, fout))


def _fps_kernel(pos8_ref, pos_ref, x_ref, pos_sel_ref, x_sel_ref, dist_ref,
                *, n, m, nc):
    colidx = (lax.broadcasted_iota(jnp.int32, (8, nc), 0) * nc
              + lax.broadcasted_iota(jnp.int32, (8, nc), 1))
    xs = pos8_ref[0:8, :]
    ys = pos8_ref[8:16, :]
    zs = pos8_ref[16:24, :]
    dist_ref[...] = jnp.full((8, nc), jnp.inf, jnp.float32)

    def body(it, cur):
        pos_sel_ref[pl.ds(it, 1), :] = pos_ref[pl.ds(cur, 1), :]
        x_sel_ref[pl.ds(it, 1), :] = x_ref[pl.ds(cur, 1), :]
        px = pos_ref[cur, 0]
        py = pos_ref[cur, 1]
        pz = pos_ref[cur, 2]
        d = (xs - px) ** 2 + (ys - py) ** 2 + (zs - pz) ** 2
        dist = jnp.minimum(dist_ref[...], d)
        dist_ref[...] = dist
        mx = jnp.max(dist)
        return jnp.min(jnp.where(dist == mx, colidx, n)).astype(jnp.int32)

    lax.fori_loop(0, m, body, jnp.int32(0))


def _fps(pos_pad, posT, x, m):
    n, f = x.shape
    nc = n // 8
    pos8 = jnp.concatenate([posT[0].reshape(8, nc), posT[1].reshape(8, nc),
                            posT[2].reshape(8, nc)], axis=0)
    body = functools.partial(_fps_kernel, n=n, m=m, nc=nc)
    return pl.pallas_call(
        body,
        in_specs=[
            pl.BlockSpec((24, nc), lambda: (0, 0)),
            pl.BlockSpec((n, 8), lambda: (0, 0)),
            pl.BlockSpec((n, f), lambda: (0, 0)),
        ],
        out_specs=[
            pl.BlockSpec((m, 8), lambda: (0, 0)),
            pl.BlockSpec((m, f), lambda: (0, 0)),
        ],
        out_shape=[jax.ShapeDtypeStruct((m, 8), jnp.float32),
                   jax.ShapeDtypeStruct((m, f), jnp.float32)],
        scratch_shapes=[pltpu.VMEM((8, nc), jnp.float32)],
        interpret=_INTERPRET,
    )(pos8, pos_pad, x)


def _head_kernel(x_ref, w1_ref, b1_ref, w2_ref, b2_ref, w3_ref, b3_ref,
                 out_ref, *, n):
    h = jnp.sum(x_ref[...], axis=0, keepdims=True) * (1.0 / n)
    h = _elu(_dot(h, w1_ref[...]) + b1_ref[...])
    h = _elu(_dot(h, w2_ref[...]) + b2_ref[...])
    o = _dot(h, w3_ref[...]) + b3_ref[...]
    mx = jnp.max(o)
    out_ref[...] = o - mx - jnp.log(jnp.sum(jnp.exp(o - mx)))


def _head(x, lw1, lb1, lw2, lb2, lw3, lb3):
    n, f = x.shape
    body = functools.partial(_head_kernel, n=n)
    return pl.pallas_call(
        body,
        out_shape=jax.ShapeDtypeStruct((1, 10), jnp.float32),
        interpret=_INTERPRET,
    )(x, lw1, lb1.reshape(1, -1), lw2, lb2.reshape(1, -1),
      lw3, lb3.reshape(1, -1))


def _pad8(p):
    n = p.shape[0]
    return jnp.concatenate([p, jnp.zeros((n, 5), jnp.float32)], axis=1)


def kernel(pos, batch, W1, root1, b1, W2, root2, b2, W3, root3, b3,
           lw1, lb1, lw2, lb2, lw3, lb3):
    n = pos.shape[0]
    pos_pad = _pad8(pos)
    posT = pos_pad.T

    perm0 = jnp.argsort(pos[:, 2])
    rank0 = jnp.argsort(perm0)
    pos_pad_s = pos_pad[perm0]
    ones = jnp.ones((n, 1), jnp.float32)
    x1 = _spline_layer(pos_pad_s, pos_pad_s.T, ones, W1, root1, b1, r=0.2,
                       xw_const=W1[:, 0, :])[rank0]

    m1 = n // 2
    pos1_pad, x1s = _fps(pos_pad, posT, x1, m1)

    perm1 = jnp.argsort(pos1_pad[:, 2])
    rank1 = jnp.argsort(perm1)
    pos1_pad_s = pos1_pad[perm1]
    x2 = _spline_layer(pos1_pad_s, pos1_pad_s.T, x1s[perm1], W2, root2, b2,
                       r=0.4)[rank1]

    m2 = m1 // 4
    pos2_pad, x2s = _fps(pos1_pad, pos1_pad.T, x2, m2)

    x3 = _spline_layer(pos2_pad, pos2_pad.T, x2s, W3, root3, b3, r=1.0)

    return _head(x3, lw1, lb1, lw2, lb2, lw3, lb3)


# speedup vs baseline: 1.7791x; 1.0004x over previous
import functools

import jax
import jax.numpy as jnp
from jax import lax
from jax.experimental import pallas as pl
from jax.experimental.pallas import tpu as pltpu

def _dot(a, b):
    return lax.dot_general(a, b, (((1,), (0,)), ((), ())),
                           preferred_element_type=jnp.float32)


def _elu(x):
    return jnp.where(x > 0, x, jnp.exp(jnp.minimum(x, 0.0)) - 1.0)


def _xw_kernel(x_ref, w_ref, out_ref):
    out_ref[0] = _dot(x_ref[...], w_ref[0]).astype(jnp.bfloat16)


def _xw(x, W):
    kk, fin, fout = W.shape
    n = x.shape[0]
    return pl.pallas_call(
        _xw_kernel,
        grid=(kk,),
        in_specs=[
            pl.BlockSpec((n, fin), lambda k: (0, 0)),
            pl.BlockSpec((1, fin, fout), lambda k: (k, 0, 0)),
        ],
        out_specs=pl.BlockSpec((1, n, fout), lambda k: (k, 0, 0)),
        out_shape=jax.ShapeDtypeStruct((kk, n, fout), jnp.bfloat16),
    )(x, W)


def _pair_geometry(pos_d_ref, pos_sT_ref, i, j, ti, tj, r):
    pd = pos_d_ref[...]
    ps = pos_sT_ref[...]
    dx = pd[:, 0:1] - ps[0:1, :]
    dy = pd[:, 1:2] - ps[1:2, :]
    dz = pd[:, 2:3] - ps[2:3, :]
    d2 = dx * dx + dy * dy + dz * dz
    ig = i * ti + lax.broadcasted_iota(jnp.int32, (ti, tj), 0)
    jg = j * tj + lax.broadcasted_iota(jnp.int32, (ti, tj), 1)
    maskf = ((d2 < r * r) & (ig != jg)).astype(jnp.float32)
    c = lax.broadcasted_iota(jnp.int32, (5, ti, tj), 0).astype(jnp.float32)
    bs = []
    for dd in (dx, dy, dz):
        v = jnp.clip(dd * (0.5 / r) + 0.5, 0.0, 1.0) * 4.0
        bs.append(jnp.maximum(1.0 - jnp.abs(v[None] - c), 0.0))
    bs[0] = bs[0] * maskf[None]
    return maskf, bs


def _layer_kernel(pos_d_ref, pos_sT_ref, xw_ref, xd_ref, root_ref, b_ref,
                  out_ref, acc_ref, cnt_ref, *, r, ti, tj, j_tiles, o_dim):
    i = pl.program_id(0)
    j = pl.program_id(1)

    @pl.when(j == 0)
    def _():
        acc_ref[...] = jnp.zeros_like(acc_ref)
        cnt_ref[...] = jnp.zeros_like(cnt_ref)

    zd = pos_d_ref[...][:, 2:3]
    zs = pos_sT_ref[...][2:3, :]
    gap = jnp.maximum(jnp.min(zd) - jnp.max(zs), jnp.min(zs) - jnp.max(zd))

    @pl.when(gap < r)
    def _():
        maskf, (b0, b1, b2) = _pair_geometry(pos_d_ref, pos_sT_ref,
                                             i, j, ti, tj, r)
        cnt_ref[...] += jnp.sum(maskf, axis=1, keepdims=True)

        b0 = b0.astype(jnp.bfloat16)
        b1 = b1.astype(jnp.bfloat16)
        b2 = b2.astype(jnp.bfloat16)
        acc = jnp.zeros((ti, o_dim), jnp.float32)
        for k0 in range(5):
            for k1 in range(5):
                p01 = b0[k0] * b1[k1]
                a5 = jnp.concatenate([p01 * b2[k2] for k2 in range(5)], axis=1)
                x5 = xw_ref[k0 * 25 + k1 * 5:k0 * 25 + k1 * 5 + 5]
                acc += _dot(a5, x5.reshape(5 * tj, o_dim))
        acc_ref[...] += acc

    @pl.when(j == j_tiles - 1)
    def _():
        agg = acc_ref[...] / jnp.maximum(cnt_ref[...], 1.0)
        res = agg + _dot(xd_ref[...], root_ref[...]) + b_ref[...]
        out_ref[...] = _elu(res)


def _spline_layer(pos_pad, posT, x, W, root, b, r, xw_const=None,
                  ti=256, tj=128):
    n = pos_pad.shape[0]
    fin, fout = root.shape
    i_tiles, j_tiles = n // ti, n // tj
    if xw_const is not None:
        xw = jnp.broadcast_to(xw_const.astype(jnp.bfloat16)[:, None, :],
                              (125, tj, fout))
        xw_spec = pl.BlockSpec((125, tj, fout), lambda i, j: (0, 0, 0))
    else:
        xw = _xw(x, W)
        xw_spec = pl.BlockSpec((125, tj, fout), lambda i, j: (0, j, 0))
    body = functools.partial(_layer_kernel, r=r, ti=ti, tj=tj,
                             j_tiles=j_tiles, o_dim=fout)
    return pl.pallas_call(
        body,
        grid=(i_tiles, j_tiles),
        in_specs=[
            pl.BlockSpec((ti, 8), lambda i, j: (i, 0)),
            pl.BlockSpec((8, tj), lambda i, j: (0, j)),
            xw_spec,
            pl.BlockSpec((ti, fin), lambda i, j: (i, 0)),
            pl.BlockSpec((fin, fout), lambda i, j: (0, 0)),
            pl.BlockSpec((1, fout), lambda i, j: (0, 0)),
        ],
        out_specs=pl.BlockSpec((ti, fout), lambda i, j: (i, 0)),
        out_shape=jax.ShapeDtypeStruct((n, fout), jnp.float32),
        scratch_shapes=[pltpu.VMEM((ti, fout), jnp.float32),
                        pltpu.VMEM((ti, 1), jnp.float32)],
        compiler_params=pltpu.CompilerParams(
            dimension_semantics=("arbitrary", "arbitrary")),
    )(pos_pad, posT, xw, x, root, b.reshape(1, fout))


def _fps_kernel(pos8_ref, pos_ref, x_ref, pos_sel_ref, x_sel_ref, dist_ref,
                *, n, m, nc):
    colidx = (lax.broadcasted_iota(jnp.int32, (8, nc), 0) * nc
              + lax.broadcasted_iota(jnp.int32, (8, nc), 1))
    xs = pos8_ref[0:8, :]
    ys = pos8_ref[8:16, :]
    zs = pos8_ref[16:24, :]
    dist_ref[...] = jnp.full((8, nc), jnp.inf, jnp.float32)

    def body(it, cur):
        pos_sel_ref[pl.ds(it, 1), :] = pos_ref[pl.ds(cur, 1), :]
        x_sel_ref[pl.ds(it, 1), :] = x_ref[pl.ds(cur, 1), :]
        px = pos_ref[cur, 0]
        py = pos_ref[cur, 1]
        pz = pos_ref[cur, 2]
        d = (xs - px) ** 2 + (ys - py) ** 2 + (zs - pz) ** 2
        dist = jnp.minimum(dist_ref[...], d)
        dist_ref[...] = dist
        mx = jnp.max(dist)
        return jnp.min(jnp.where(dist == mx, colidx, n)).astype(jnp.int32)

    lax.fori_loop(0, m, body, jnp.int32(0))


def _fps(pos_pad, posT, x, m):
    n, f = x.shape
    nc = n // 8
    pos8 = jnp.concatenate([posT[0].reshape(8, nc), posT[1].reshape(8, nc),
                            posT[2].reshape(8, nc)], axis=0)
    body = functools.partial(_fps_kernel, n=n, m=m, nc=nc)
    return pl.pallas_call(
        body,
        in_specs=[
            pl.BlockSpec((24, nc), lambda: (0, 0)),
            pl.BlockSpec((n, 8), lambda: (0, 0)),
            pl.BlockSpec((n, f), lambda: (0, 0)),
        ],
        out_specs=[
            pl.BlockSpec((m, 8), lambda: (0, 0)),
            pl.BlockSpec((m, f), lambda: (0, 0)),
        ],
        out_shape=[jax.ShapeDtypeStruct((m, 8), jnp.float32),
                   jax.ShapeDtypeStruct((m, f), jnp.float32)],
        scratch_shapes=[pltpu.VMEM((8, nc), jnp.float32)],
    )(pos8, pos_pad, x)


def _head_kernel(x_ref, w1_ref, b1_ref, w2_ref, b2_ref, w3_ref, b3_ref,
                 out_ref, *, n):
    h = jnp.sum(x_ref[...], axis=0, keepdims=True) * (1.0 / n)
    h = _elu(_dot(h, w1_ref[...]) + b1_ref[...])
    h = _elu(_dot(h, w2_ref[...]) + b2_ref[...])
    o = _dot(h, w3_ref[...]) + b3_ref[...]
    mx = jnp.max(o)
    out_ref[...] = o - mx - jnp.log(jnp.sum(jnp.exp(o - mx)))


def _head(x, lw1, lb1, lw2, lb2, lw3, lb3):
    n, f = x.shape
    body = functools.partial(_head_kernel, n=n)
    return pl.pallas_call(
        body,
        out_shape=jax.ShapeDtypeStruct((1, 10), jnp.float32),
    )(x, lw1, lb1.reshape(1, -1), lw2, lb2.reshape(1, -1),
      lw3, lb3.reshape(1, -1))


def _pad8(p):
    n = p.shape[0]
    return jnp.concatenate([p, jnp.zeros((n, 5), jnp.float32)], axis=1)


def kernel(pos, batch, W1, root1, b1, W2, root2, b2, W3, root3, b3,
           lw1, lb1, lw2, lb2, lw3, lb3):
    n = pos.shape[0]
    pos_pad = _pad8(pos)
    posT = pos_pad.T

    perm0 = jnp.argsort(pos[:, 2])
    rank0 = jnp.argsort(perm0)
    pos_pad_s = pos_pad[perm0]
    ones = jnp.ones((n, 1), jnp.float32)
    x1 = _spline_layer(pos_pad_s, pos_pad_s.T, ones, W1, root1, b1, r=0.2,
                       xw_const=W1[:, 0, :])[rank0]

    m1 = n // 2
    pos1_pad, x1s = _fps(pos_pad, posT, x1, m1)

    perm1 = jnp.argsort(pos1_pad[:, 2])
    rank1 = jnp.argsort(perm1)
    pos1_pad_s = pos1_pad[perm1]
    x2 = _spline_layer(pos1_pad_s, pos1_pad_s.T, x1s[perm1], W2, root2, b2,
                       r=0.4)[rank1]

    m2 = m1 // 4
    pos2_pad, x2s = _fps(pos1_pad, pos1_pad.T, x2, m2)

    x3 = _spline_layer(pos2_pad, pos2_pad.T, x2s, W3, root3, b3, r=1.0)

    return _head(x3, lw1, lb1, lw2, lb2, lw3, lb3)
